# Initial kernel scaffold; baseline (speedup 1.0000x reference)
import jax
import jax.numpy as jnp
from jax.experimental import pallas as pl


def kernel(boxes, scores):
    raise NotImplementedError("write your pallas kernel here")


# speedup vs baseline: 21.7506x; 21.7506x over previous
import functools

import jax
import jax.numpy as jnp
from jax import lax
from jax.experimental import pallas as pl

_N = 20000
_MAX_OUT = 100
_IOU_THRESH = 0.5
_ROWS = 160
_COLS = 128
_NP = _ROWS * _COLS


def _nms_body(x1_ref, y1_ref, x2_ref, y2_ref, sc_ref, out_ref):
    x1 = x1_ref[...]
    y1 = y1_ref[...]
    x2 = x2_ref[...]
    y2 = y2_ref[...]
    sc0 = sc_ref[...]
    a2 = (x2 - x1) * (y2 - y1)
    ridx = lax.broadcasted_iota(jnp.int32, (_ROWS, _COLS), 0)
    cidx = lax.broadcasted_iota(jnp.int32, (_ROWS, _COLS), 1)
    idx = ridx * _COLS + cidx
    col1 = lax.broadcasted_iota(jnp.int32, (1, _COLS), 1)
    neg_inf = jnp.float32(-jnp.inf)

    def it(i, sc):
        m = jnp.max(sc)
        j = jnp.min(jnp.where(sc == m, idx, _NP))
        sel = idx == j
        zero = jnp.float32(0.0)
        x1b = jnp.sum(jnp.where(sel, x1, zero))
        y1b = jnp.sum(jnp.where(sel, y1, zero))
        x2b = jnp.sum(jnp.where(sel, x2, zero))
        y2b = jnp.sum(jnp.where(sel, y2, zero))
        sb = jnp.sum(jnp.where(sel, sc0, zero))
        a1 = (x2b - x1b) * (y2b - y1b)
        ix1 = jnp.maximum(x1b, x1)
        iy1 = jnp.maximum(y1b, y1)
        ix2 = jnp.minimum(x2b, x2)
        iy2 = jnp.minimum(y2b, y2)
        iw = jnp.maximum(ix2 - ix1, zero)
        ih = jnp.maximum(iy2 - iy1, zero)
        inter = iw * ih
        union = (a1 + a2) - inter
        iou = inter / (union + jnp.float32(1e-9))
        sc = jnp.where(iou > _IOU_THRESH, neg_inf, sc)
        row = jnp.where(col1 == 0, x1b,
              jnp.where(col1 == 1, y1b,
              jnp.where(col1 == 2, x2b,
              jnp.where(col1 == 3, y2b,
              jnp.where(col1 == 4, sb, zero)))))
        out_ref[pl.ds(i, 1), :] = row
        return sc

    lax.fori_loop(0, _MAX_OUT, it, sc0)


@jax.jit
def kernel(boxes, scores):
    pad = _NP - _N
    x1 = jnp.pad(boxes[:, 0], (0, pad)).reshape(_ROWS, _COLS)
    y1 = jnp.pad(boxes[:, 1], (0, pad)).reshape(_ROWS, _COLS)
    x2 = jnp.pad(boxes[:, 2], (0, pad), constant_values=1.0).reshape(_ROWS, _COLS)
    y2 = jnp.pad(boxes[:, 3], (0, pad), constant_values=1.0).reshape(_ROWS, _COLS)
    sc = jnp.pad(scores, (0, pad), constant_values=-jnp.inf).reshape(_ROWS, _COLS)
    out = pl.pallas_call(
        _nms_body,
        out_shape=jax.ShapeDtypeStruct((_MAX_OUT, _COLS), jnp.float32),
    )(x1, y1, x2, y2, sc)
    return out[:, :5]
